# baseline (speedup 1.0000x reference)
import functools

import jax
import jax.numpy as jnp
from jax import lax
from jax.experimental import pallas as pl
from jax.experimental.pallas import tpu as pltpu
from jax.experimental.pallas import tpu_sc as plsc

E = 320000
D = 128
C = 128
NW = 32
CPW = 80
EPW = C * CPW
E_PAD = NW * EPW

_mesh = plsc.VectorSubcoreMesh(core_axis_name="c", subcore_axis_name="s")


@functools.partial(
    pl.kernel,
    out_type=jax.ShapeDtypeStruct((E_PAD,), jnp.float32),
    mesh=_mesh,
    scratch_types=[
        pltpu.VMEM((C,), jnp.int32),
        pltpu.VMEM((C,), jnp.int32),
        pltpu.VMEM((C, D), jnp.float32),
        pltpu.VMEM((C, D), jnp.float32),
        pltpu.VMEM((C,), jnp.float32),
        pltpu.SemaphoreType.DMA,
    ],
    compiler_params=pltpu.CompilerParams(needs_layout_passes=False),
)
def _sddmm(src_idx_hbm, dst_idx_hbm, src_feat_hbm, dst_feat_hbm, out_hbm,
           sidx_v, didx_v, srows_v, drows_v, out_v, sem):
    wid = lax.axis_index("s") * 2 + lax.axis_index("c")
    base = wid * EPW

    def chunk_body(ci, carry):
        off = base + ci * C
        pltpu.sync_copy(src_idx_hbm.at[pl.ds(off, C)], sidx_v)
        pltpu.sync_copy(dst_idx_hbm.at[pl.ds(off, C)], didx_v)
        cp_s = pltpu.async_copy(src_feat_hbm.at[sidx_v], srows_v, sem)
        cp_d = pltpu.async_copy(dst_feat_hbm.at[didx_v], drows_v, sem)
        cp_s.wait()
        cp_d.wait()

        def g_body(g, c2):
            rows = g * 16 + lax.iota(jnp.int32, 16)
            acc = jnp.zeros((16,), jnp.float32)
            for d in range(D):
                col = jnp.full((16,), d, jnp.int32)
                sv = plsc.load_gather(srows_v, [rows, col])
                dv = plsc.load_gather(drows_v, [rows, col])
                acc = acc + sv * dv
            out_v[pl.ds(g * 16, 16)] = acc
            return c2

        lax.fori_loop(0, C // 16, g_body, 0)
        pltpu.sync_copy(out_v, out_hbm.at[pl.ds(off, C)])
        return carry

    lax.fori_loop(0, CPW, chunk_body, 0)


def kernel(src_idx, dst_idx, src_feat, dst_feat):
    pad = E_PAD - E
    src_idx_p = jnp.concatenate(
        [src_idx.astype(jnp.int32), jnp.zeros((pad,), jnp.int32)])
    dst_idx_p = jnp.concatenate(
        [dst_idx.astype(jnp.int32), jnp.zeros((pad,), jnp.int32)])
    out = _sddmm(src_idx_p, dst_idx_p, src_feat, dst_feat)
    return out[:E]


# speedup vs baseline: 2.1185x; 2.1185x over previous
import functools

import jax
import jax.numpy as jnp
from jax import lax
from jax.experimental import pallas as pl
from jax.experimental.pallas import tpu as pltpu
from jax.experimental.pallas import tpu_sc as plsc

E = 320000
D = 128
C = 128
NW = 32
CPW = 80
EPW = C * CPW
E_PAD = NW * EPW

_mesh = plsc.VectorSubcoreMesh(core_axis_name="c", subcore_axis_name="s")


@functools.partial(
    pl.kernel,
    out_type=jax.ShapeDtypeStruct((E_PAD,), jnp.float32),
    mesh=_mesh,
    scratch_types=[
        pltpu.VMEM((C,), jnp.int32),
        pltpu.VMEM((C,), jnp.int32),
        pltpu.VMEM((C, D), jnp.float32),
        pltpu.VMEM((C, D), jnp.float32),
        pltpu.VMEM((C,), jnp.float32),
        pltpu.SemaphoreType.DMA,
    ],
    compiler_params=pltpu.CompilerParams(needs_layout_passes=False),
)
def _sddmm(src_idx_hbm, dst_idx_hbm, src_feat_hbm, dst_feat_hbm, out_hbm,
           sidx_v, didx_v, srows_v, drows_v, out_v, sem):
    wid = lax.axis_index("s") * 2 + lax.axis_index("c")
    base = wid * EPW

    def chunk_body(ci, carry):
        off = base + ci * C
        pltpu.sync_copy(src_idx_hbm.at[pl.ds(off, C)], sidx_v)
        pltpu.sync_copy(dst_idx_hbm.at[pl.ds(off, C)], didx_v)
        cp_s = pltpu.async_copy(src_feat_hbm.at[sidx_v], srows_v, sem)
        cp_d = pltpu.async_copy(dst_feat_hbm.at[didx_v], drows_v, sem)
        cp_s.wait()
        cp_d.wait()

        def g_body(g, c2):
            lane = lax.iota(jnp.int32, 16)
            rows = g * 16 + lane
            acc = jnp.zeros((16,), jnp.float32)
            for d in range(D):
                col = jnp.bitwise_and(lane + d, D - 1)
                sv = plsc.load_gather(srows_v, [rows, col])
                dv = plsc.load_gather(drows_v, [rows, col])
                acc = acc + sv * dv
            out_v[pl.ds(g * 16, 16)] = acc
            return c2

        lax.fori_loop(0, C // 16, g_body, 0)
        pltpu.sync_copy(out_v, out_hbm.at[pl.ds(off, C)])
        return carry

    lax.fori_loop(0, CPW, chunk_body, 0)


def kernel(src_idx, dst_idx, src_feat, dst_feat):
    pad = E_PAD - E
    src_idx_p = jnp.concatenate(
        [src_idx.astype(jnp.int32), jnp.zeros((pad,), jnp.int32)])
    dst_idx_p = jnp.concatenate(
        [dst_idx.astype(jnp.int32), jnp.zeros((pad,), jnp.int32)])
    out = _sddmm(src_idx_p, dst_idx_p, src_feat, dst_feat)
    return out[:E]


# speedup vs baseline: 2.6288x; 1.2409x over previous
import functools

import jax
import jax.numpy as jnp
from jax import lax
from jax.experimental import pallas as pl
from jax.experimental.pallas import tpu as pltpu
from jax.experimental.pallas import tpu_sc as plsc

E = 320000
D = 128
C = 160
NW = 32
NCH = 64
EPW = C * NCH
E_PAD = NW * EPW

_mesh = plsc.VectorSubcoreMesh(core_axis_name="c", subcore_axis_name="s")


@functools.partial(
    pl.kernel,
    out_type=jax.ShapeDtypeStruct((E_PAD,), jnp.float32),
    mesh=_mesh,
    scratch_types=[
        pltpu.VMEM((C,), jnp.int32),
        pltpu.VMEM((C,), jnp.int32),
        pltpu.VMEM((C,), jnp.int32),
        pltpu.VMEM((C,), jnp.int32),
        pltpu.VMEM((2, C, D), jnp.float32),
        pltpu.VMEM((2, C, D), jnp.float32),
        pltpu.VMEM((C,), jnp.float32),
        pltpu.VMEM((C,), jnp.float32),
        pltpu.SemaphoreType.DMA,
        pltpu.SemaphoreType.DMA,
        pltpu.SemaphoreType.DMA,
        pltpu.SemaphoreType.DMA,
        pltpu.SemaphoreType.DMA,
        pltpu.SemaphoreType.DMA,
    ],
    compiler_params=pltpu.CompilerParams(needs_layout_passes=False),
)
def _sddmm(src_idx_hbm, dst_idx_hbm, src_feat_hbm, dst_feat_hbm, out_hbm,
           sidx0, sidx1, didx0, didx1, srows_v, drows_v, out0, out1,
           sem_i0, sem_i1, sem_g0, sem_g1, sem_o0, sem_o1):
    sidx = (sidx0, sidx1)
    didx = (didx0, didx1)
    outb = (out0, out1)
    sem_i = (sem_i0, sem_i1)
    sem_g = (sem_g0, sem_g1)
    sem_o = (sem_o0, sem_o1)
    wid = lax.axis_index("s") * 2 + lax.axis_index("c")
    base = wid * EPW
    last = NCH - 1

    def fire_idx(b, c):
        off = base + jnp.minimum(c, last) * C
        pltpu.async_copy(src_idx_hbm.at[pl.ds(off, C)], sidx[b], sem_i[b])
        pltpu.async_copy(dst_idx_hbm.at[pl.ds(off, C)], didx[b], sem_i[b])

    def wait_idx(b):
        pltpu.make_async_copy(
            src_idx_hbm.at[pl.ds(base, C)], sidx[b], sem_i[b]).wait()
        pltpu.make_async_copy(
            dst_idx_hbm.at[pl.ds(base, C)], didx[b], sem_i[b]).wait()

    def fire_gather(b):
        pltpu.async_copy(src_feat_hbm.at[sidx[b]], srows_v.at[b], sem_g[b])
        pltpu.async_copy(dst_feat_hbm.at[didx[b]], drows_v.at[b], sem_g[b])

    def wait_gather(b):
        pltpu.make_async_copy(
            src_feat_hbm.at[sidx[b]], srows_v.at[b], sem_g[b]).wait()
        pltpu.make_async_copy(
            dst_feat_hbm.at[didx[b]], drows_v.at[b], sem_g[b]).wait()

    def wait_out(b):
        pltpu.make_async_copy(
            outb[b], out_hbm.at[pl.ds(base, C)], sem_o[b]).wait()

    fire_idx(0, 0)
    wait_idx(0)
    fire_gather(0)
    fire_idx(1, 1)

    def s_body(s, carry):
        for b in range(2):
            c = s * 2 + b
            wait_gather(b)
            wait_idx(1 - b)
            fire_gather(1 - b)
            fire_idx(b, c + 2)

            sb = srows_v.at[b]
            db = drows_v.at[b]

            @pl.when(s > 0)
            def _():
                wait_out(b)

            def g_body(g, c2):
                lane = lax.iota(jnp.int32, 16)
                rows = g * 16 + lane
                acc = jnp.zeros((16,), jnp.float32)
                for d in range(D):
                    col = jnp.bitwise_and(lane + d, D - 1)
                    sv = plsc.load_gather(sb, [rows, col])
                    dv = plsc.load_gather(db, [rows, col])
                    acc = acc + sv * dv
                outb[b][pl.ds(g * 16, 16)] = acc
                return c2

            lax.fori_loop(0, C // 16, g_body, 0)
            pltpu.async_copy(
                outb[b], out_hbm.at[pl.ds(base + c * C, C)], sem_o[b])
        return carry

    lax.fori_loop(0, NCH // 2, s_body, 0)

    wait_gather(0)
    wait_idx(1)
    wait_out(0)
    wait_out(1)


def kernel(src_idx, dst_idx, src_feat, dst_feat):
    pad = E_PAD - E
    src_idx_p = jnp.concatenate(
        [src_idx.astype(jnp.int32), jnp.zeros((pad,), jnp.int32)])
    dst_idx_p = jnp.concatenate(
        [dst_idx.astype(jnp.int32), jnp.zeros((pad,), jnp.int32)])
    out = _sddmm(src_idx_p, dst_idx_p, src_feat, dst_feat)
    return out[:E]


# speedup vs baseline: 5.8562x; 2.2277x over previous
import functools

import jax
import jax.numpy as jnp
from jax import lax
from jax.experimental import pallas as pl
from jax.experimental.pallas import tpu as pltpu
from jax.experimental.pallas import tpu_sc as plsc

E = 320000
D = 128
N = 10000
NT = 16
Q = D // (2 * NT)
TW = N * Q
TWP = 40064
CE = 1024
E_PAD = 327680
EH = E_PAD // 2
NCH = EH // CE
SW = EH // NT

_mesh = plsc.VectorSubcoreMesh(core_axis_name="c", subcore_axis_name="s")


@functools.partial(
    pl.kernel,
    out_type=(
        jax.ShapeDtypeStruct((E_PAD,), jnp.float32),
        jax.ShapeDtypeStruct((NT * E_PAD,), jnp.float32),
    ),
    mesh=_mesh,
    scratch_types=[
        pltpu.VMEM((TWP,), jnp.int32),
        pltpu.VMEM((TWP,), jnp.int32),
        pltpu.VMEM((CE,), jnp.int32),
        pltpu.VMEM((CE,), jnp.int32),
        pltpu.VMEM((CE,), jnp.int32),
        pltpu.VMEM((CE,), jnp.int32),
        pltpu.VMEM((CE,), jnp.float32),
        pltpu.VMEM((CE,), jnp.float32),
        pltpu.VMEM((NT, CE), jnp.float32),
        pltpu.VMEM((CE,), jnp.float32),
        pltpu.SemaphoreType.DMA,
        pltpu.SemaphoreType.DMA,
        pltpu.SemaphoreType.DMA,
        pltpu.SemaphoreType.DMA,
        pltpu.SemaphoreType.DMA,
        pltpu.SemaphoreType.DMA,
    ],
    compiler_params=pltpu.CompilerParams(needs_layout_passes=False),
)
def _sddmm(src_idx_hbm, dst_idx_hbm, st_hbm, dt_hbm, out_hbm, par_hbm,
           st, dt, sidx0, sidx1, didx0, didx1, par0, par1, red, accv,
           sem_i0, sem_i1, sem_p0, sem_p1, sem_t, sem_r):
    sidx = (sidx0, sidx1)
    didx = (didx0, didx1)
    par = (par0, par1)
    sem_i = (sem_i0, sem_i1)
    sem_p = (sem_p0, sem_p1)
    cid = lax.axis_index("c")
    tid = lax.axis_index("s")
    half = cid * EH
    pbase = tid * E_PAD + half
    last = NCH - 1

    pltpu.async_copy(st_hbm.at[pl.ds(tid * TWP, TWP)], st, sem_t)
    pltpu.async_copy(dt_hbm.at[pl.ds(tid * TWP, TWP)], dt, sem_t)

    def fire_idx(b, c):
        off = half + jnp.minimum(c, last) * CE
        pltpu.async_copy(src_idx_hbm.at[pl.ds(off, CE)], sidx[b], sem_i[b])
        pltpu.async_copy(dst_idx_hbm.at[pl.ds(off, CE)], didx[b], sem_i[b])

    def wait_idx(b):
        pltpu.make_async_copy(
            src_idx_hbm.at[pl.ds(half, CE)], sidx[b], sem_i[b]).wait()
        pltpu.make_async_copy(
            dst_idx_hbm.at[pl.ds(half, CE)], didx[b], sem_i[b]).wait()

    def wait_par(b):
        pltpu.make_async_copy(
            par[b], par_hbm.at[pl.ds(pbase, CE)], sem_p[b]).wait()

    fire_idx(0, 0)
    fire_idx(1, 1)
    pltpu.make_async_copy(st_hbm.at[pl.ds(0, TWP)], st, sem_t).wait()
    pltpu.make_async_copy(dt_hbm.at[pl.ds(0, TWP)], dt, sem_t).wait()

    lane = lax.iota(jnp.int32, 16)
    himask = jnp.full((16,), -65536, jnp.int32)
    colrot = tuple(jnp.bitwise_and(lane + q, Q - 1) for q in range(Q))

    def s_body(s, carry):
        for b in range(2):
            c = s * 2 + b
            wait_idx(b)

            @pl.when(s > 0)
            def _():
                wait_par(b)

            def g_body(g, c2):
                sg = sidx[b][pl.ds(g * 16, 16)] * Q
                dg = didx[b][pl.ds(g * 16, 16)] * Q
                acc = jnp.zeros((16,), jnp.float32)
                for q in range(Q):
                    sv = plsc.load_gather(st, [sg + colrot[q]])
                    dv = plsc.load_gather(dt, [dg + colrot[q]])
                    s_lo = plsc.bitcast(lax.shift_left(sv, 16), jnp.float32)
                    d_lo = plsc.bitcast(lax.shift_left(dv, 16), jnp.float32)
                    s_hi = plsc.bitcast(
                        jnp.bitwise_and(sv, himask), jnp.float32)
                    d_hi = plsc.bitcast(
                        jnp.bitwise_and(dv, himask), jnp.float32)
                    acc = acc + s_lo * d_lo
                    acc = acc + s_hi * d_hi
                par[b][pl.ds(g * 16, 16)] = acc
                return c2

            lax.fori_loop(0, CE // 16, g_body, 0)
            pltpu.async_copy(
                par[b], par_hbm.at[pl.ds(pbase + c * CE, CE)], sem_p[b])
            fire_idx(b, c + 2)
        return carry

    lax.fori_loop(0, NCH // 2, s_body, 0)

    wait_idx(0)
    wait_idx(1)
    wait_par(0)
    wait_par(1)
    plsc.subcore_barrier()

    sbase = half + tid * SW
    for sub in range(SW // CE):
        off = sbase + sub * CE
        for t in range(NT):
            pltpu.async_copy(
                par_hbm.at[pl.ds(t * E_PAD + off, CE)], red.at[t], sem_r)
        for t in range(NT):
            pltpu.make_async_copy(
                par_hbm.at[pl.ds(0, CE)], red.at[t], sem_r).wait()

        def r_body(v, c2):
            acc = red[0, pl.ds(v * 16, 16)]
            for t in range(1, NT):
                acc = acc + red[t, pl.ds(v * 16, 16)]
            accv[pl.ds(v * 16, 16)] = acc
            return c2

        lax.fori_loop(0, CE // 16, r_body, 0)
        pltpu.sync_copy(accv, out_hbm.at[pl.ds(off, CE)])


def kernel(src_idx, dst_idx, src_feat, dst_feat):
    pad = E_PAD - E
    src_idx_p = jnp.concatenate(
        [src_idx.astype(jnp.int32), jnp.zeros((pad,), jnp.int32)])
    dst_idx_p = jnp.concatenate(
        [dst_idx.astype(jnp.int32), jnp.zeros((pad,), jnp.int32)])

    def shard(feat):
        pk = lax.bitcast_convert_type(
            feat.astype(jnp.bfloat16).reshape(N, D // 2, 2), jnp.int32)
        sl = pk.reshape(N, NT, Q).transpose(1, 0, 2).reshape(NT, TW)
        sl = jnp.pad(sl, ((0, 0), (0, TWP - TW)))
        return sl.reshape(NT * TWP)

    out, _ = _sddmm(src_idx_p, dst_idx_p, shard(src_feat), shard(dst_feat))
    return out[:E]
